# baseline (speedup 1.0000x reference)
import functools

import jax
import jax.numpy as jnp
from jax import lax
from jax.experimental import pallas as pl
from jax.experimental.pallas import tpu as pltpu
from jax.experimental.pallas import tpu_sc as plsc

_MAX_LEN = 200
_D_MODEL = 64
_BATCH = 4096
_ROW = _MAX_LEN * _D_MODEL

_NUM_CORES = 2
_NUM_SUBCORES = 16
_NUM_WORKERS = _NUM_CORES * _NUM_SUBCORES
_ROWS_PER_W = _BATCH // _NUM_WORKERS

_REP = 8
_CHUNKS = _ROWS_PER_W // _REP


@functools.partial(
    pl.kernel,
    out_type=jax.ShapeDtypeStruct((_BATCH, _ROW), jnp.float32),
    mesh=plsc.VectorSubcoreMesh(core_axis_name="c", subcore_axis_name="s"),
    scratch_types=[
        pltpu.VMEM((_REP, _ROW), jnp.float32),
        pltpu.SemaphoreType.DMA,
    ],
)
def _pe_broadcast(w_hbm, out_hbm, buf, sem):
    wid = lax.axis_index("c") * _NUM_SUBCORES + lax.axis_index("s")
    base = wid * _ROWS_PER_W
    for r in range(_REP):
        pltpu.sync_copy(w_hbm, buf.at[r])
    copies = [
        pltpu.async_copy(buf, out_hbm.at[pl.ds(base + i * _REP, _REP)], sem)
        for i in range(_CHUNKS)
    ]
    for c in copies:
        c.wait()


def kernel(x, pe_weight):
    del x
    out = _pe_broadcast(pe_weight.reshape(_ROW))
    return out.reshape(_BATCH, _MAX_LEN, _D_MODEL)
